# baseline (speedup 1.0000x reference)
import functools

import jax
import jax.numpy as jnp
from jax import lax
from jax.experimental import pallas as pl
from jax.experimental.pallas import tpu as pltpu
from jax.experimental.pallas import tpu_sc as plsc

DIM = 64
MARGIN = 1.0
C_COEF = 1.0
CHUNK = 128
NC = 2
NS = 16
NW = NC * NS


def _sc_gather_pairs(tab2, rtab2, nvtab2, qe3, qr3):
    cpw = qe3.shape[1]
    n_ent = NW * cpw * CHUNK
    n_rel = NW * CHUNK

    mesh = plsc.VectorSubcoreMesh(core_axis_name="c", subcore_axis_name="s")
    out_type = (
        jax.ShapeDtypeStruct((n_ent, 2 * DIM), jnp.float32),
        jax.ShapeDtypeStruct((n_rel, 2 * DIM), jnp.float32),
        jax.ShapeDtypeStruct((n_rel, 2 * DIM), jnp.float32),
    )

    @functools.partial(
        pl.kernel,
        mesh=mesh,
        out_type=out_type,
        scratch_types=[
            pltpu.VMEM((cpw, CHUNK), jnp.int32),
            pltpu.VMEM((1, CHUNK), jnp.int32),
            pltpu.VMEM((CHUNK, 2 * DIM), jnp.float32),
            pltpu.VMEM((CHUNK, 2 * DIM), jnp.float32),
            pltpu.SemaphoreType.DMA,
            pltpu.SemaphoreType.DMA,
        ],
    )
    def gather_k(tab, rtab, nvtab, qe_h, qr_h, ent_o, r_o, nv_o,
                 idx_v, ridx_v, buf0, buf1, sem0, sem1):
        wid = lax.axis_index("s") * NC + lax.axis_index("c")
        cbase = wid * cpw
        pltpu.sync_copy(qe_h.at[wid], idx_v)

        bufs = (buf0, buf1)
        sems = (sem0, sem1)

        pltpu.async_copy(tab.at[idx_v.at[0]], buf0, sem0)

        def body(k, carry):
            j0 = 2 * k
            pltpu.async_copy(tab.at[idx_v.at[j0 + 1]], bufs[1], sems[1])
            pltpu.make_async_copy(tab.at[idx_v.at[j0]], bufs[0], sems[0]).wait()
            pltpu.sync_copy(bufs[0], ent_o.at[pl.ds((cbase + j0) * CHUNK, CHUNK)])

            @pl.when(k < (cpw // 2) - 1)
            def _():
                pltpu.async_copy(tab.at[idx_v.at[j0 + 2]], bufs[0], sems[0])

            pltpu.make_async_copy(
                tab.at[idx_v.at[j0 + 1]], bufs[1], sems[1]).wait()
            pltpu.sync_copy(
                bufs[1], ent_o.at[pl.ds((cbase + j0 + 1) * CHUNK, CHUNK)])
            return carry

        lax.fori_loop(0, cpw // 2, body, 0)

        pltpu.sync_copy(qr_h.at[wid], ridx_v)
        pltpu.async_copy(rtab.at[ridx_v.at[0]], buf0, sem0).wait()
        pltpu.sync_copy(buf0, r_o.at[pl.ds(wid * CHUNK, CHUNK)])
        pltpu.async_copy(nvtab.at[ridx_v.at[0]], buf1, sem1).wait()
        pltpu.sync_copy(buf1, nv_o.at[pl.ds(wid * CHUNK, CHUNK)])

    return gather_k(tab2, rtab2, nvtab2, qe3, qr3)


def _tc_loss(epairs, rpairs, nvpairs, pf, B, NEG):
    ngrid = B // CHUNK
    negblk = CHUNK * NEG

    def sel(p2, par):
        lo = p2[:, :DIM]
        hi = p2[:, DIM:]
        return lo + par * (hi - lo)

    def body(h_r, t_r, nh_r, nt_r, rr_r, nv_r, pf_r, loss_r,
             acc_m, acc_s, acc_o, acc_r):
        i = pl.program_id(0)

        @pl.when(i == 0)
        def _init():
            acc_m[0, 0] = 0.0
            acc_s[0, 0] = 0.0
            acc_o[0, 0] = 0.0
            acc_r[0, 0] = 0.0

        par = pf_r[...]
        h = sel(h_r[...], par[:, 0:1])
        t = sel(t_r[...], par[:, 1:2])
        r = sel(rr_r[...], par[:, 2:3])
        nv_raw = sel(nv_r[...], par[:, 2:3])

        denom = jnp.maximum(
            jnp.sum(jnp.abs(nv_raw), axis=1, keepdims=True), 1e-12)
        nv = nv_raw / denom
        d = h - t
        dot = jnp.sum(d * nv, axis=1, keepdims=True)
        e = d - dot * nv + r
        pos = jnp.sum(jnp.abs(e), axis=1, keepdims=True)

        nh3 = nh_r[...].reshape(CHUNK, NEG, 2 * DIM)
        nt3 = nt_r[...].reshape(CHUNK, NEG, 2 * DIM)
        pnh = par[:, 3:3 + NEG][:, :, None]
        pnt = par[:, 3 + NEG:3 + 2 * NEG][:, :, None]
        nh = nh3[:, :, :DIM] + pnh * (nh3[:, :, DIM:] - nh3[:, :, :DIM])
        nt = nt3[:, :, :DIM] + pnt * (nt3[:, :, DIM:] - nt3[:, :, :DIM])

        dd = nh - nt
        nvu = nv[:, None, :]
        ndot = jnp.sum(dd * nvu, axis=2, keepdims=True)
        ne = dd - ndot * nvu + r[:, None, :]
        ndist = jnp.sum(jnp.abs(ne), axis=2)

        acc_m[0, 0] += jnp.sum(jnp.maximum(pos + MARGIN - ndist, 0.0))
        acc_s[0, 0] += (
            jnp.sum(jnp.maximum(jnp.sum(h * h, axis=1) - 1.0, 0.0))
            + jnp.sum(jnp.maximum(jnp.sum(t * t, axis=1) - 1.0, 0.0))
            + jnp.sum(jnp.maximum(jnp.sum(nh * nh, axis=2) - 1.0, 0.0))
            + jnp.sum(jnp.maximum(jnp.sum(nt * nt, axis=2) - 1.0, 0.0)))
        acc_o[0, 0] += jnp.sum(jnp.sum(nv * r, axis=1) ** 2)
        acc_r[0, 0] += jnp.sum(jnp.maximum(jnp.sum(r * r, axis=1) - 1.0, 0.0))

        @pl.when(i == ngrid - 1)
        def _fin():
            n_embs = 2.0 * B + 2.0 * B * NEG
            loss_r[0, 0] = (
                acc_m[0, 0] / (B * NEG)
                + C_COEF * (acc_o[0, 0] / B
                            + acc_s[0, 0] / n_embs
                            + acc_r[0, 0] / B))

    out = pl.pallas_call(
        body,
        grid=(ngrid,),
        in_specs=[
            pl.BlockSpec((CHUNK, 2 * DIM), lambda i: (i, 0)),
            pl.BlockSpec((CHUNK, 2 * DIM), lambda i: (i + ngrid, 0)),
            pl.BlockSpec((negblk, 2 * DIM),
                         lambda i: (i + (2 * B) // negblk, 0)),
            pl.BlockSpec((negblk, 2 * DIM),
                         lambda i: (i + (2 * B + B * NEG) // negblk, 0)),
            pl.BlockSpec((CHUNK, 2 * DIM), lambda i: (i, 0)),
            pl.BlockSpec((CHUNK, 2 * DIM), lambda i: (i, 0)),
            pl.BlockSpec((CHUNK, 2 * DIM), lambda i: (i, 0)),
        ],
        out_specs=pl.BlockSpec(memory_space=pltpu.SMEM),
        out_shape=jax.ShapeDtypeStruct((1, 1), jnp.float32),
        scratch_shapes=[pltpu.SMEM((1, 1), jnp.float32)] * 4,
    )(epairs, epairs, epairs, epairs, rpairs, nvpairs, pf)
    return out[0, 0]


def kernel(h, r, t, neg_samples, entity_emb, relation_emb, norm_vector_table):
    B = h.shape[0]
    NEG = neg_samples.shape[1]
    eidx = jnp.concatenate([
        h, t,
        neg_samples[:, :, 0].reshape(-1),
        neg_samples[:, :, 1].reshape(-1),
    ])
    qe = eidx >> 1
    pe = (eidx & 1).astype(jnp.float32)
    qr = r >> 1
    pr = (r & 1).astype(jnp.float32)

    tab2 = entity_emb.reshape(-1, 2 * DIM)
    rtab2 = relation_emb.reshape(-1, 2 * DIM)
    nvtab2 = norm_vector_table.reshape(-1, 2 * DIM)

    pf = jnp.concatenate([
        pe[:B][:, None], pe[B:2 * B][:, None], pr[:, None],
        pe[2 * B:2 * B + B * NEG].reshape(B, NEG),
        pe[2 * B + B * NEG:].reshape(B, NEG),
        jnp.zeros((B, 2 * DIM - 3 - 2 * NEG), jnp.float32),
    ], axis=1)

    qe3 = qe.reshape(NW, -1, CHUNK)
    qr3 = qr.reshape(NW, 1, CHUNK)
    epairs, rpairs, nvpairs = _sc_gather_pairs(tab2, rtab2, nvtab2, qe3, qr3)
    return _tc_loss(epairs, rpairs, nvpairs, pf, B, NEG)


# speedup vs baseline: 1.6317x; 1.6317x over previous
import functools

import jax
import jax.numpy as jnp
from jax import lax
from jax.experimental import pallas as pl
from jax.experimental.pallas import tpu as pltpu
from jax.experimental.pallas import tpu_sc as plsc

DIM = 64
MARGIN = 1.0
C_COEF = 1.0
CHUNK = 128
NC = 2
NS = 16
NW = NC * NS
PBLK = 7296
NBLK = 69
PAIRH = PBLK * NBLK


def _tc_pack(et):
    def body(a_r, b_r, o_r):
        o_r[...] = jnp.concatenate([a_r[...].T, b_r[...].T], axis=1)

    return pl.pallas_call(
        body,
        grid=(NBLK,),
        in_specs=[
            pl.BlockSpec((DIM, PBLK), lambda i: (0, i)),
            pl.BlockSpec((DIM, PBLK), lambda i: (0, i + NBLK)),
        ],
        out_specs=pl.BlockSpec((PBLK, 2 * DIM), lambda i: (i, 0)),
        out_shape=jax.ShapeDtypeStruct((PAIRH, 2 * DIM), jnp.float32),
    )(et, et)


def _sc_gather_pairs(tab2, rtab2, nvtab2, qe3, qr3):
    cpw = qe3.shape[1]
    n_ent = NW * cpw * CHUNK
    n_rel = NW * CHUNK

    mesh = plsc.VectorSubcoreMesh(core_axis_name="c", subcore_axis_name="s")
    out_type = (
        jax.ShapeDtypeStruct((n_ent, 2 * DIM), jnp.float32),
        jax.ShapeDtypeStruct((n_rel, 2 * DIM), jnp.float32),
        jax.ShapeDtypeStruct((n_rel, 2 * DIM), jnp.float32),
    )

    @functools.partial(
        pl.kernel,
        mesh=mesh,
        out_type=out_type,
        scratch_types=[
            pltpu.VMEM((cpw, CHUNK), jnp.int32),
            pltpu.VMEM((1, CHUNK), jnp.int32),
            pltpu.VMEM((CHUNK, 2 * DIM), jnp.float32),
            pltpu.VMEM((CHUNK, 2 * DIM), jnp.float32),
            pltpu.SemaphoreType.DMA,
            pltpu.SemaphoreType.DMA,
        ],
    )
    def gather_k(tab, rtab, nvtab, qe_h, qr_h, ent_o, r_o, nv_o,
                 idx_v, ridx_v, buf0, buf1, sem0, sem1):
        wid = lax.axis_index("s") * NC + lax.axis_index("c")
        cbase = wid * cpw
        pltpu.sync_copy(qe_h.at[wid], idx_v)

        bufs = (buf0, buf1)
        sems = (sem0, sem1)

        pltpu.async_copy(tab.at[idx_v.at[0]], buf0, sem0)

        def body(k, carry):
            j0 = 2 * k
            pltpu.async_copy(tab.at[idx_v.at[j0 + 1]], bufs[1], sems[1])
            pltpu.make_async_copy(tab.at[idx_v.at[j0]], bufs[0], sems[0]).wait()
            pltpu.sync_copy(bufs[0], ent_o.at[pl.ds((cbase + j0) * CHUNK, CHUNK)])

            @pl.when(k < (cpw // 2) - 1)
            def _():
                pltpu.async_copy(tab.at[idx_v.at[j0 + 2]], bufs[0], sems[0])

            pltpu.make_async_copy(
                tab.at[idx_v.at[j0 + 1]], bufs[1], sems[1]).wait()
            pltpu.sync_copy(
                bufs[1], ent_o.at[pl.ds((cbase + j0 + 1) * CHUNK, CHUNK)])
            return carry

        lax.fori_loop(0, cpw // 2, body, 0)

        pltpu.sync_copy(qr_h.at[wid], ridx_v)
        pltpu.async_copy(rtab.at[ridx_v.at[0]], buf0, sem0).wait()
        pltpu.sync_copy(buf0, r_o.at[pl.ds(wid * CHUNK, CHUNK)])
        pltpu.async_copy(nvtab.at[ridx_v.at[0]], buf1, sem1).wait()
        pltpu.sync_copy(buf1, nv_o.at[pl.ds(wid * CHUNK, CHUNK)])

    return gather_k(tab2, rtab2, nvtab2, qe3, qr3)


def _tc_loss(epairs, rpairs, nvpairs, pf, B, NEG):
    ngrid = B // CHUNK
    negblk = CHUNK * NEG

    def sel(p2, par):
        lo = p2[:, :DIM]
        hi = p2[:, DIM:]
        return lo + par * (hi - lo)

    def body(h_r, t_r, nh_r, nt_r, rr_r, nv_r, pf_r, loss_r,
             acc_m, acc_s, acc_o, acc_r):
        i = pl.program_id(0)

        @pl.when(i == 0)
        def _init():
            acc_m[0, 0] = 0.0
            acc_s[0, 0] = 0.0
            acc_o[0, 0] = 0.0
            acc_r[0, 0] = 0.0

        par = pf_r[...]
        h = sel(h_r[...], par[:, 0:1])
        t = sel(t_r[...], par[:, 1:2])
        r = sel(rr_r[...], par[:, 2:3])
        nv_raw = sel(nv_r[...], par[:, 2:3])

        denom = jnp.maximum(
            jnp.sum(jnp.abs(nv_raw), axis=1, keepdims=True), 1e-12)
        nv = nv_raw / denom
        d = h - t
        dot = jnp.sum(d * nv, axis=1, keepdims=True)
        e = d - dot * nv + r
        pos = jnp.sum(jnp.abs(e), axis=1, keepdims=True)

        nh3 = nh_r[...].reshape(CHUNK, NEG, 2 * DIM)
        nt3 = nt_r[...].reshape(CHUNK, NEG, 2 * DIM)
        pnh = par[:, 3:3 + NEG][:, :, None]
        pnt = par[:, 3 + NEG:3 + 2 * NEG][:, :, None]
        nh = nh3[:, :, :DIM] + pnh * (nh3[:, :, DIM:] - nh3[:, :, :DIM])
        nt = nt3[:, :, :DIM] + pnt * (nt3[:, :, DIM:] - nt3[:, :, :DIM])

        dd = nh - nt
        nvu = nv[:, None, :]
        ndot = jnp.sum(dd * nvu, axis=2, keepdims=True)
        ne = dd - ndot * nvu + r[:, None, :]
        ndist = jnp.sum(jnp.abs(ne), axis=2)

        acc_m[0, 0] += jnp.sum(jnp.maximum(pos + MARGIN - ndist, 0.0))
        acc_s[0, 0] += (
            jnp.sum(jnp.maximum(jnp.sum(h * h, axis=1) - 1.0, 0.0))
            + jnp.sum(jnp.maximum(jnp.sum(t * t, axis=1) - 1.0, 0.0))
            + jnp.sum(jnp.maximum(jnp.sum(nh * nh, axis=2) - 1.0, 0.0))
            + jnp.sum(jnp.maximum(jnp.sum(nt * nt, axis=2) - 1.0, 0.0)))
        acc_o[0, 0] += jnp.sum(jnp.sum(nv * r, axis=1) ** 2)
        acc_r[0, 0] += jnp.sum(jnp.maximum(jnp.sum(r * r, axis=1) - 1.0, 0.0))

        @pl.when(i == ngrid - 1)
        def _fin():
            n_embs = 2.0 * B + 2.0 * B * NEG
            loss_r[0, 0] = (
                acc_m[0, 0] / (B * NEG)
                + C_COEF * (acc_o[0, 0] / B
                            + acc_s[0, 0] / n_embs
                            + acc_r[0, 0] / B))

    out = pl.pallas_call(
        body,
        grid=(ngrid,),
        in_specs=[
            pl.BlockSpec((CHUNK, 2 * DIM), lambda i: (i, 0)),
            pl.BlockSpec((CHUNK, 2 * DIM), lambda i: (i + ngrid, 0)),
            pl.BlockSpec((negblk, 2 * DIM),
                         lambda i: (i + (2 * B) // negblk, 0)),
            pl.BlockSpec((negblk, 2 * DIM),
                         lambda i: (i + (2 * B + B * NEG) // negblk, 0)),
            pl.BlockSpec((CHUNK, 2 * DIM), lambda i: (i, 0)),
            pl.BlockSpec((CHUNK, 2 * DIM), lambda i: (i, 0)),
            pl.BlockSpec((CHUNK, 2 * DIM), lambda i: (i, 0)),
        ],
        out_specs=pl.BlockSpec(memory_space=pltpu.SMEM),
        out_shape=jax.ShapeDtypeStruct((1, 1), jnp.float32),
        scratch_shapes=[pltpu.SMEM((1, 1), jnp.float32)] * 4,
    )(epairs, epairs, epairs, epairs, rpairs, nvpairs, pf)
    return out[0, 0]


def kernel(h, r, t, neg_samples, entity_emb, relation_emb, norm_vector_table):
    B = h.shape[0]
    NEG = neg_samples.shape[1]
    eidx = jnp.concatenate([
        h, t,
        neg_samples[:, :, 0].reshape(-1),
        neg_samples[:, :, 1].reshape(-1),
    ])
    pe_b = eidx >= PAIRH
    qe = jnp.where(pe_b, eidx - PAIRH, eidx)
    pe = pe_b.astype(jnp.float32)
    qr = r >> 1
    pr = (r & 1).astype(jnp.float32)

    tab2 = _tc_pack(entity_emb.T)
    rtab2 = relation_emb.reshape(-1, 2 * DIM)
    nvtab2 = norm_vector_table.reshape(-1, 2 * DIM)

    pf = jnp.concatenate([
        pe[:B][:, None], pe[B:2 * B][:, None], pr[:, None],
        pe[2 * B:2 * B + B * NEG].reshape(B, NEG),
        pe[2 * B + B * NEG:].reshape(B, NEG),
        jnp.zeros((B, 2 * DIM - 3 - 2 * NEG), jnp.float32),
    ], axis=1)

    qe3 = qe.reshape(NW, -1, CHUNK)
    qr3 = qr.reshape(NW, 1, CHUNK)
    epairs, rpairs, nvpairs = _sc_gather_pairs(tab2, rtab2, nvtab2, qe3, qr3)
    return _tc_loss(epairs, rpairs, nvpairs, pf, B, NEG)


# speedup vs baseline: 1.7509x; 1.0730x over previous
import functools

import jax
import jax.numpy as jnp
from jax import lax
from jax.experimental import pallas as pl
from jax.experimental.pallas import tpu as pltpu
from jax.experimental.pallas import tpu_sc as plsc

DIM = 64
MARGIN = 1.0
C_COEF = 1.0
CHUNK = 128
NC = 2
NS = 16
NW = NC * NS
PBLK = 7296
NBLK = 69
PAIRH = PBLK * NBLK


def _tc_pack(et):
    def body(a_r, b_r, o_r):
        o_r[...] = jnp.concatenate([a_r[...].T, b_r[...].T], axis=1)

    return pl.pallas_call(
        body,
        grid=(NBLK,),
        in_specs=[
            pl.BlockSpec((DIM, PBLK), lambda i: (0, i)),
            pl.BlockSpec((DIM, PBLK), lambda i: (0, i + NBLK)),
        ],
        out_specs=pl.BlockSpec((PBLK, 2 * DIM), lambda i: (i, 0)),
        out_shape=jax.ShapeDtypeStruct((PAIRH, 2 * DIM), jnp.float32),
    )(et, et)


def _sc_gather_pairs(tab2, rtab2, nvtab2, qe3, qr3):
    cpw = qe3.shape[1]
    n_ent = NW * cpw * CHUNK
    n_rel = NW * CHUNK
    with_rel = qr3 is not None

    mesh = plsc.VectorSubcoreMesh(core_axis_name="c", subcore_axis_name="s")
    if with_rel:
        out_type = (
            jax.ShapeDtypeStruct((n_ent, 2 * DIM), jnp.float32),
            jax.ShapeDtypeStruct((n_rel, 2 * DIM), jnp.float32),
            jax.ShapeDtypeStruct((n_rel, 2 * DIM), jnp.float32),
        )
    else:
        out_type = jax.ShapeDtypeStruct((n_ent, 2 * DIM), jnp.float32)

    @functools.partial(
        pl.kernel,
        mesh=mesh,
        out_type=out_type,
        scratch_types=[
            pltpu.VMEM((cpw, CHUNK), jnp.int32),
            pltpu.VMEM((1, CHUNK), jnp.int32),
            pltpu.VMEM((CHUNK, 2 * DIM), jnp.float32),
            pltpu.VMEM((CHUNK, 2 * DIM), jnp.float32),
            pltpu.SemaphoreType.DMA,
            pltpu.SemaphoreType.DMA,
        ],
    )
    def gather_k(*refs):
        if with_rel:
            (tab, rtab, nvtab, qe_h, qr_h, ent_o, r_o, nv_o,
             idx_v, ridx_v, buf0, buf1, sem0, sem1) = refs
        else:
            (tab, qe_h, ent_o,
             idx_v, ridx_v, buf0, buf1, sem0, sem1) = refs
        wid = lax.axis_index("s") * NC + lax.axis_index("c")
        cbase = wid * cpw
        pltpu.sync_copy(qe_h.at[wid], idx_v)

        bufs = (buf0, buf1)
        sems = (sem0, sem1)

        pltpu.async_copy(tab.at[idx_v.at[0]], buf0, sem0)

        def body(k, carry):
            j0 = 2 * k
            pltpu.async_copy(tab.at[idx_v.at[j0 + 1]], bufs[1], sems[1])
            pltpu.make_async_copy(tab.at[idx_v.at[j0]], bufs[0], sems[0]).wait()
            pltpu.sync_copy(bufs[0], ent_o.at[pl.ds((cbase + j0) * CHUNK, CHUNK)])

            @pl.when(k < (cpw // 2) - 1)
            def _():
                pltpu.async_copy(tab.at[idx_v.at[j0 + 2]], bufs[0], sems[0])

            pltpu.make_async_copy(
                tab.at[idx_v.at[j0 + 1]], bufs[1], sems[1]).wait()
            pltpu.sync_copy(
                bufs[1], ent_o.at[pl.ds((cbase + j0 + 1) * CHUNK, CHUNK)])
            return carry

        lax.fori_loop(0, cpw // 2, body, 0)

        if cpw % 2:
            j = cpw - 1
            pltpu.async_copy(tab.at[idx_v.at[j]], buf0, sem0).wait()
            pltpu.sync_copy(buf0, ent_o.at[pl.ds((cbase + j) * CHUNK, CHUNK)])

        if with_rel:
            pltpu.sync_copy(qr_h.at[wid], ridx_v)
            pltpu.async_copy(rtab.at[ridx_v.at[0]], buf0, sem0).wait()
            pltpu.sync_copy(buf0, r_o.at[pl.ds(wid * CHUNK, CHUNK)])
            pltpu.async_copy(nvtab.at[ridx_v.at[0]], buf1, sem1).wait()
            pltpu.sync_copy(buf1, nv_o.at[pl.ds(wid * CHUNK, CHUNK)])

    if with_rel:
        return gather_k(tab2, rtab2, nvtab2, qe3, qr3)
    return (gather_k(tab2, qe3),)


def _tc_loss(epairs, rpairs, nvpairs, pf, pnh_a, pnt_a, B, NEG, rbase):
    ngrid = B // CHUNK
    negblk = CHUNK * NEG

    def sel(p2, par):
        lo = p2[:, :DIM]
        hi = p2[:, DIM:]
        return lo + par * (hi - lo)

    def body(h_r, t_r, nh_r, nt_r, rr_r, nv_r, pf_r, pnh_r, pnt_r, loss_r,
             acc_m, acc_s, acc_o, acc_r):
        i = pl.program_id(0)

        @pl.when(i == 0)
        def _init():
            acc_m[0, 0] = 0.0
            acc_s[0, 0] = 0.0
            acc_o[0, 0] = 0.0
            acc_r[0, 0] = 0.0

        par = pf_r[...]
        h = sel(h_r[...], par[:, 0:1])
        t = sel(t_r[...], par[:, 1:2])
        r = sel(rr_r[...], par[:, 2:3])
        nv_raw = sel(nv_r[...], par[:, 2:3])

        denom = jnp.maximum(
            jnp.sum(jnp.abs(nv_raw), axis=1, keepdims=True), 1e-12)
        nv = nv_raw / denom
        d = h - t
        dot = jnp.sum(d * nv, axis=1, keepdims=True)
        e = d - dot * nv + r
        pos = jnp.sum(jnp.abs(e), axis=1, keepdims=True)

        nh3 = nh_r[...].reshape(CHUNK, NEG, 2 * DIM)
        nt3 = nt_r[...].reshape(CHUNK, NEG, 2 * DIM)
        pnh = pnh_r[...][:, :, None]
        pnt = pnt_r[...][:, :, None]
        nh = nh3[:, :, :DIM] + pnh * (nh3[:, :, DIM:] - nh3[:, :, :DIM])
        nt = nt3[:, :, :DIM] + pnt * (nt3[:, :, DIM:] - nt3[:, :, :DIM])

        dd = nh - nt
        nvu = nv[:, None, :]
        ndot = jnp.sum(dd * nvu, axis=2, keepdims=True)
        ne = dd - ndot * nvu + r[:, None, :]
        ndist = jnp.sum(jnp.abs(ne), axis=2)

        acc_m[0, 0] += jnp.sum(jnp.maximum(pos + MARGIN - ndist, 0.0))
        acc_s[0, 0] += (
            jnp.sum(jnp.maximum(jnp.sum(h * h, axis=1) - 1.0, 0.0))
            + jnp.sum(jnp.maximum(jnp.sum(t * t, axis=1) - 1.0, 0.0))
            + jnp.sum(jnp.maximum(jnp.sum(nh * nh, axis=2) - 1.0, 0.0))
            + jnp.sum(jnp.maximum(jnp.sum(nt * nt, axis=2) - 1.0, 0.0)))
        acc_o[0, 0] += jnp.sum(jnp.sum(nv * r, axis=1) ** 2)
        acc_r[0, 0] += jnp.sum(jnp.maximum(jnp.sum(r * r, axis=1) - 1.0, 0.0))

        @pl.when(i == ngrid - 1)
        def _fin():
            loss_r[0, 0] = acc_m[0, 0]
            loss_r[0, 1] = acc_s[0, 0]
            loss_r[0, 2] = acc_o[0, 0]
            loss_r[0, 3] = acc_r[0, 0]

    return pl.pallas_call(
        body,
        grid=(ngrid,),
        in_specs=[
            pl.BlockSpec((CHUNK, 2 * DIM), lambda i: (i, 0)),
            pl.BlockSpec((CHUNK, 2 * DIM), lambda i: (i + ngrid, 0)),
            pl.BlockSpec((negblk, 2 * DIM),
                         lambda i: (i + (2 * B) // negblk, 0)),
            pl.BlockSpec((negblk, 2 * DIM),
                         lambda i: (i + (2 * B + B * NEG) // negblk, 0)),
            pl.BlockSpec((CHUNK, 2 * DIM), lambda i: (i + rbase, 0)),
            pl.BlockSpec((CHUNK, 2 * DIM), lambda i: (i + rbase, 0)),
            pl.BlockSpec((CHUNK, 2 * DIM), lambda i: (i, 0)),
            pl.BlockSpec((CHUNK, NEG), lambda i: (i, 0)),
            pl.BlockSpec((CHUNK, NEG), lambda i: (i, 0)),
        ],
        out_specs=pl.BlockSpec(memory_space=pltpu.SMEM),
        out_shape=jax.ShapeDtypeStruct((1, 4), jnp.float32),
        scratch_shapes=[pltpu.SMEM((1, 1), jnp.float32)] * 4,
    )(epairs, epairs, epairs, epairs, rpairs, nvpairs, pf, pnh_a, pnt_a)


def kernel(h, r, t, neg_samples, entity_emb, relation_emb, norm_vector_table):
    B = h.shape[0]
    NEG = neg_samples.shape[1]
    Bh = B // 2
    qr = r >> 1
    pr = (r & 1).astype(jnp.float32)

    tab2 = _tc_pack(entity_emb.T)
    rtab2 = relation_emb.reshape(-1, 2 * DIM)
    nvtab2 = norm_vector_table.reshape(-1, 2 * DIM)

    def half(lo, hi):
        eidx = jnp.concatenate([
            h[lo:hi], t[lo:hi],
            neg_samples[lo:hi, :, 0].reshape(-1),
            neg_samples[lo:hi, :, 1].reshape(-1),
        ])
        pe_b = eidx >= PAIRH
        qe = jnp.where(pe_b, eidx - PAIRH, eidx)
        pe = pe_b.astype(jnp.float32)
        pf = jnp.concatenate([
            pe[:Bh][:, None], pe[Bh:2 * Bh][:, None], pr[lo:hi][:, None],
            jnp.zeros((Bh, 2 * DIM - 3), jnp.float32),
        ], axis=1)
        pnh_a = pe[2 * Bh:2 * Bh + Bh * NEG].reshape(Bh, NEG)
        pnt_a = pe[2 * Bh + Bh * NEG:].reshape(Bh, NEG)
        return qe.reshape(NW, -1, CHUNK), pf, pnh_a, pnt_a

    qe3_a, pf_a, pnh_a, pnt_a = half(0, Bh)
    qe3_b, pf_b, pnh_b, pnt_b = half(Bh, B)
    qr3 = qr.reshape(NW, 1, CHUNK)

    epairs_a, rpairs, nvpairs = _sc_gather_pairs(tab2, rtab2, nvtab2,
                                                 qe3_a, qr3)
    (epairs_b,) = _sc_gather_pairs(tab2, None, None, qe3_b, None)
    sums_a = _tc_loss(epairs_a, rpairs, nvpairs, pf_a, pnh_a, pnt_a,
                      Bh, NEG, 0)
    sums_b = _tc_loss(epairs_b, rpairs, nvpairs, pf_b, pnh_b, pnt_b,
                      Bh, NEG, Bh // CHUNK)
    s = sums_a + sums_b
    n_embs = 2.0 * B + 2.0 * B * NEG
    return (s[0, 0] / (B * NEG)
            + C_COEF * (s[0, 2] / B + s[0, 1] / n_embs + s[0, 3] / B))


# speedup vs baseline: 1.7526x; 1.0010x over previous
import functools

import jax
import jax.numpy as jnp
from jax import lax
from jax.experimental import pallas as pl
from jax.experimental.pallas import tpu as pltpu
from jax.experimental.pallas import tpu_sc as plsc

DIM = 64
MARGIN = 1.0
C_COEF = 1.0
CHUNK = 128
NC = 2
NS = 16
NW = NC * NS
PBLK = 7296
NBLK = 69
PAIRH = PBLK * NBLK


def _tc_pack(et):
    def body(a_r, b_r, o_r):
        o_r[:, :DIM] = a_r[...].T
        o_r[:, DIM:] = b_r[...].T

    return pl.pallas_call(
        body,
        grid=(NBLK,),
        in_specs=[
            pl.BlockSpec((DIM, PBLK), lambda i: (0, i)),
            pl.BlockSpec((DIM, PBLK), lambda i: (0, i + NBLK)),
        ],
        out_specs=pl.BlockSpec((PBLK, 2 * DIM), lambda i: (i, 0)),
        out_shape=jax.ShapeDtypeStruct((PAIRH, 2 * DIM), jnp.float32),
    )(et, et)


def _sc_gather_pairs(tab2, rtab2, nvtab2, qe3, qr3):
    cpw = qe3.shape[1]
    n_ent = NW * cpw * CHUNK
    n_rel = NW * CHUNK
    with_rel = qr3 is not None

    mesh = plsc.VectorSubcoreMesh(core_axis_name="c", subcore_axis_name="s")
    if with_rel:
        out_type = (
            jax.ShapeDtypeStruct((n_ent, 2 * DIM), jnp.float32),
            jax.ShapeDtypeStruct((n_rel, 2 * DIM), jnp.float32),
            jax.ShapeDtypeStruct((n_rel, 2 * DIM), jnp.float32),
        )
    else:
        out_type = jax.ShapeDtypeStruct((n_ent, 2 * DIM), jnp.float32)

    @functools.partial(
        pl.kernel,
        mesh=mesh,
        out_type=out_type,
        scratch_types=[
            pltpu.VMEM((cpw, CHUNK), jnp.int32),
            pltpu.VMEM((1, CHUNK), jnp.int32),
            pltpu.VMEM((CHUNK, 2 * DIM), jnp.float32),
            pltpu.VMEM((CHUNK, 2 * DIM), jnp.float32),
            pltpu.SemaphoreType.DMA,
            pltpu.SemaphoreType.DMA,
        ],
    )
    def gather_k(*refs):
        if with_rel:
            (tab, rtab, nvtab, qe_h, qr_h, ent_o, r_o, nv_o,
             idx_v, ridx_v, buf0, buf1, sem0, sem1) = refs
        else:
            (tab, qe_h, ent_o,
             idx_v, ridx_v, buf0, buf1, sem0, sem1) = refs
        wid = lax.axis_index("s") * NC + lax.axis_index("c")
        cbase = wid * cpw
        pltpu.sync_copy(qe_h.at[wid], idx_v)

        bufs = (buf0, buf1)
        sems = (sem0, sem1)

        pltpu.async_copy(tab.at[idx_v.at[0]], buf0, sem0)

        def body(k, carry):
            j0 = 2 * k
            pltpu.async_copy(tab.at[idx_v.at[j0 + 1]], bufs[1], sems[1])
            pltpu.make_async_copy(tab.at[idx_v.at[j0]], bufs[0], sems[0]).wait()
            pltpu.sync_copy(bufs[0], ent_o.at[pl.ds((cbase + j0) * CHUNK, CHUNK)])

            @pl.when(k < (cpw // 2) - 1)
            def _():
                pltpu.async_copy(tab.at[idx_v.at[j0 + 2]], bufs[0], sems[0])

            pltpu.make_async_copy(
                tab.at[idx_v.at[j0 + 1]], bufs[1], sems[1]).wait()
            pltpu.sync_copy(
                bufs[1], ent_o.at[pl.ds((cbase + j0 + 1) * CHUNK, CHUNK)])
            return carry

        lax.fori_loop(0, cpw // 2, body, 0)

        if cpw % 2:
            j = cpw - 1
            pltpu.async_copy(tab.at[idx_v.at[j]], buf0, sem0).wait()
            pltpu.sync_copy(buf0, ent_o.at[pl.ds((cbase + j) * CHUNK, CHUNK)])

        if with_rel:
            pltpu.sync_copy(qr_h.at[wid], ridx_v)
            pltpu.async_copy(rtab.at[ridx_v.at[0]], buf0, sem0).wait()
            pltpu.sync_copy(buf0, r_o.at[pl.ds(wid * CHUNK, CHUNK)])
            pltpu.async_copy(nvtab.at[ridx_v.at[0]], buf1, sem1).wait()
            pltpu.sync_copy(buf1, nv_o.at[pl.ds(wid * CHUNK, CHUNK)])

    if with_rel:
        return gather_k(tab2, rtab2, nvtab2, qe3, qr3)
    return (gather_k(tab2, qe3),)


def _tc_loss(epairs, rpairs, nvpairs, pf, pnh_a, pnt_a, B, NEG, rbase):
    ngrid = B // CHUNK
    negblk = CHUNK * NEG

    def sel(p2, par):
        lo = p2[:, :DIM]
        hi = p2[:, DIM:]
        return lo + par * (hi - lo)

    def body(h_r, t_r, nh_r, nt_r, rr_r, nv_r, pf_r, pnh_r, pnt_r, loss_r,
             acc_m, acc_s, acc_o, acc_r):
        i = pl.program_id(0)

        @pl.when(i == 0)
        def _init():
            acc_m[0, 0] = 0.0
            acc_s[0, 0] = 0.0
            acc_o[0, 0] = 0.0
            acc_r[0, 0] = 0.0

        par = pf_r[...]
        h = sel(h_r[...], par[:, 0:1])
        t = sel(t_r[...], par[:, 1:2])
        r = sel(rr_r[...], par[:, 2:3])
        nv_raw = sel(nv_r[...], par[:, 2:3])

        denom = jnp.maximum(
            jnp.sum(jnp.abs(nv_raw), axis=1, keepdims=True), 1e-12)
        nv = nv_raw / denom
        d = h - t
        dot = jnp.sum(d * nv, axis=1, keepdims=True)
        e = d - dot * nv + r
        pos = jnp.sum(jnp.abs(e), axis=1, keepdims=True)

        nh3 = nh_r[...].reshape(CHUNK, NEG, 2 * DIM)
        nt3 = nt_r[...].reshape(CHUNK, NEG, 2 * DIM)
        pnh = pnh_r[...][:, :, None]
        pnt = pnt_r[...][:, :, None]
        nh = nh3[:, :, :DIM] + pnh * (nh3[:, :, DIM:] - nh3[:, :, :DIM])
        nt = nt3[:, :, :DIM] + pnt * (nt3[:, :, DIM:] - nt3[:, :, :DIM])

        dd = nh - nt
        nvu = nv[:, None, :]
        ndot = jnp.sum(dd * nvu, axis=2, keepdims=True)
        ne = dd - ndot * nvu + r[:, None, :]
        ndist = jnp.sum(jnp.abs(ne), axis=2)

        acc_m[0, 0] += jnp.sum(jnp.maximum(pos + MARGIN - ndist, 0.0))
        acc_s[0, 0] += (
            jnp.sum(jnp.maximum(jnp.sum(h * h, axis=1) - 1.0, 0.0))
            + jnp.sum(jnp.maximum(jnp.sum(t * t, axis=1) - 1.0, 0.0))
            + jnp.sum(jnp.maximum(jnp.sum(nh * nh, axis=2) - 1.0, 0.0))
            + jnp.sum(jnp.maximum(jnp.sum(nt * nt, axis=2) - 1.0, 0.0)))
        acc_o[0, 0] += jnp.sum(jnp.sum(nv * r, axis=1) ** 2)
        acc_r[0, 0] += jnp.sum(jnp.maximum(jnp.sum(r * r, axis=1) - 1.0, 0.0))

        @pl.when(i == ngrid - 1)
        def _fin():
            loss_r[0, 0] = acc_m[0, 0]
            loss_r[0, 1] = acc_s[0, 0]
            loss_r[0, 2] = acc_o[0, 0]
            loss_r[0, 3] = acc_r[0, 0]

    return pl.pallas_call(
        body,
        grid=(ngrid,),
        in_specs=[
            pl.BlockSpec((CHUNK, 2 * DIM), lambda i: (i, 0)),
            pl.BlockSpec((CHUNK, 2 * DIM), lambda i: (i + ngrid, 0)),
            pl.BlockSpec((negblk, 2 * DIM),
                         lambda i: (i + (2 * B) // negblk, 0)),
            pl.BlockSpec((negblk, 2 * DIM),
                         lambda i: (i + (2 * B + B * NEG) // negblk, 0)),
            pl.BlockSpec((CHUNK, 2 * DIM), lambda i: (i + rbase, 0)),
            pl.BlockSpec((CHUNK, 2 * DIM), lambda i: (i + rbase, 0)),
            pl.BlockSpec((CHUNK, 2 * DIM), lambda i: (i, 0)),
            pl.BlockSpec((CHUNK, NEG), lambda i: (i, 0)),
            pl.BlockSpec((CHUNK, NEG), lambda i: (i, 0)),
        ],
        out_specs=pl.BlockSpec(memory_space=pltpu.SMEM),
        out_shape=jax.ShapeDtypeStruct((1, 4), jnp.float32),
        scratch_shapes=[pltpu.SMEM((1, 1), jnp.float32)] * 4,
    )(epairs, epairs, epairs, epairs, rpairs, nvpairs, pf, pnh_a, pnt_a)


def kernel(h, r, t, neg_samples, entity_emb, relation_emb, norm_vector_table):
    B = h.shape[0]
    NEG = neg_samples.shape[1]
    Bh = B // 2
    qr = r >> 1
    pr = (r & 1).astype(jnp.float32)

    tab2 = _tc_pack(entity_emb.T)
    rtab2 = relation_emb.reshape(-1, 2 * DIM)
    nvtab2 = norm_vector_table.reshape(-1, 2 * DIM)

    def half(lo, hi):
        eidx = jnp.concatenate([
            h[lo:hi], t[lo:hi],
            neg_samples[lo:hi, :, 0].reshape(-1),
            neg_samples[lo:hi, :, 1].reshape(-1),
        ])
        pe_b = eidx >= PAIRH
        qe = jnp.where(pe_b, eidx - PAIRH, eidx)
        pe = pe_b.astype(jnp.float32)
        pf = jnp.concatenate([
            pe[:Bh][:, None], pe[Bh:2 * Bh][:, None], pr[lo:hi][:, None],
            jnp.zeros((Bh, 2 * DIM - 3), jnp.float32),
        ], axis=1)
        pnh_a = pe[2 * Bh:2 * Bh + Bh * NEG].reshape(Bh, NEG)
        pnt_a = pe[2 * Bh + Bh * NEG:].reshape(Bh, NEG)
        return qe.reshape(NW, -1, CHUNK), pf, pnh_a, pnt_a

    qe3_a, pf_a, pnh_a, pnt_a = half(0, Bh)
    qe3_b, pf_b, pnh_b, pnt_b = half(Bh, B)
    qr3 = qr.reshape(NW, 1, CHUNK)

    epairs_a, rpairs, nvpairs = _sc_gather_pairs(tab2, rtab2, nvtab2,
                                                 qe3_a, qr3)
    (epairs_b,) = _sc_gather_pairs(tab2, None, None, qe3_b, None)
    sums_a = _tc_loss(epairs_a, rpairs, nvpairs, pf_a, pnh_a, pnt_a,
                      Bh, NEG, 0)
    sums_b = _tc_loss(epairs_b, rpairs, nvpairs, pf_b, pnh_b, pnt_b,
                      Bh, NEG, Bh // CHUNK)
    s = sums_a + sums_b
    n_embs = 2.0 * B + 2.0 * B * NEG
    return (s[0, 0] / (B * NEG)
            + C_COEF * (s[0, 2] / B + s[0, 1] / n_embs + s[0, 3] / B))
